# baseline (speedup 1.0000x reference)
import functools

import numpy as np
import jax
import jax.numpy as jnp
from jax.experimental import pallas as pl
from jax.experimental.pallas import tpu as pltpu


_RT = 96
_CT = 1280

_A = {c: 0.1 * (c + 1) for c in range(19)}
_B = {c: 0.01 * c for c in range(19)}


def _slant_desc(dch, xch, ych, dscale, pre=1.0):
    return ('s', (_A[dch] * dscale * pre, _B[dch] * dscale * pre,
                  _A[xch] * pre, _B[xch] * pre,
                  _A[ych] * pre, _B[ych] * pre))


def _near_desc(ch, pre=1.0):
    return ('n', (_A[ch] * pre, _B[ch] * pre))


def _tu_maps(up):
    u = float(up)
    return [
        _slant_desc(0, 1, 2, u, 1.0),
        _slant_desc(0, 1, 2, u, 0.9),
        _slant_desc(16, 17, 18, u, 1.0),
        _slant_desc(16, 17, 18, u, 0.9),
        _near_desc(1, 1.0),
        _near_desc(1, 0.9),
        _near_desc(2, 1.0),
        _near_desc(2, 0.9),
        ('e', 0),
        ('e', 1),
        _near_desc(17, 1.0),
        _near_desc(17, 0.9),
        _near_desc(18, 1.0),
        _near_desc(18, 0.9),
    ]


_TU_NAMES = ['fx_cur', 'fx_pre', 'fy_cur', 'fy_pre', 'fxx_cur', 'fxx_pre',
             'fxy_cur', 'fxy_pre', 'conf_cur', 'conf_pre', 'fyx_cur',
             'fyx_pre', 'fyy_cur', 'fyy_pre']


def _plain_maps(dscale):
    return [
        _slant_desc(0, 1, 2, dscale, 1.0),
        _slant_desc(16, 17, 18, dscale, 1.0),
        _near_desc(1, 1.0),
        _near_desc(2, 1.0),
        _near_desc(17, 1.0),
        _near_desc(18, 1.0),
    ]


@functools.lru_cache(maxsize=None)
def _one_hots(up, h, w):
    H, W = h * up, w * up
    R = (np.arange(H)[:, None] // up == np.arange(h)[None, :]).astype(np.float32)
    C = (np.arange(w)[:, None] == (np.arange(W)[None, :] // up)).astype(np.float32)
    return C, R


def _make_body(maps, up, nconf, rt, ct):
    cen = (up - 1) / 2.0
    has_slant = any(k == 's' for k, _ in maps)
    euse = sorted({p for k, p in maps if k == 'e'})

    def body(*refs):
        if nconf:
            b_ref, e_ref, c_ref, r_ref = refs[:4]
            o_refs = refs[4:]
        else:
            b_ref, c_ref, r_ref = refs[:3]
            o_refs = refs[3:]
        cmat = c_ref[...]
        rmat = r_ref[...]
        z = jnp.dot(b_ref[...], cmat, preferred_element_type=jnp.float32)
        bu = jnp.dot(rmat, z, preferred_element_type=jnp.float32)
        if has_slant:
            ri = jax.lax.broadcasted_iota(jnp.int32, (rt, ct), 0)
            ci = jax.lax.broadcasted_iota(jnp.int32, (rt, ct), 1)
            dif = (ri & (up - 1)).astype(jnp.float32) - cen
            djf = (ci & (up - 1)).astype(jnp.float32) - cen
        eu = {}
        for k in euse:
            zk = jnp.dot(e_ref[k], cmat, preferred_element_type=jnp.float32)
            eu[k] = jnp.dot(rmat, zk, preferred_element_type=jnp.float32)
        for n, (kind, p) in enumerate(maps):
            if kind == 'n':
                o_refs[n][...] = bu * p[0] + p[1]
            elif kind == 's':
                ad, bd, ax, bx, ay, by = p
                o_refs[n][...] = (bu * (ad + ax * djf + ay * dif)
                                  + (bd + bx * djf + by * dif))
            else:
                o_refs[n][...] = eu[p]
    return body


def _level_call(base, econf, up, maps):
    B, _, h, w = base.shape
    H, W = h * up, w * up
    ct = min(_CT, W)
    rt = up
    for cand in (192, 96, 64, 48, 32, 16, 8):
        if (H % cand == 0 and cand % up == 0
                and len(maps) * cand * ct * 8 <= 40e6):
            rt = cand
            break
    C, R = _one_hots(up, h, w)
    nconf = 0 if econf is None else econf.shape[1]
    body = _make_body(maps, up, nconf, rt, ct)

    in_specs = [pl.BlockSpec((None, None, h, w), lambda b, r, c: (b, 0, 0, 0))]
    args = [base]
    if nconf:
        in_specs.append(
            pl.BlockSpec((None, nconf, h, w), lambda b, r, c: (b, 0, 0, 0)))
        args.append(econf)
    in_specs.append(pl.BlockSpec((w, ct), lambda b, r, c: (0, c)))
    in_specs.append(pl.BlockSpec((rt, h), lambda b, r, c: (r, 0)))
    args += [jnp.asarray(C), jnp.asarray(R)]

    n_out = len(maps)
    outs = pl.pallas_call(
        body,
        out_shape=[jax.ShapeDtypeStruct((B, 1, H, W), jnp.float32)] * n_out,
        grid=(B, H // rt, W // ct),
        in_specs=in_specs,
        out_specs=[pl.BlockSpec((None, None, rt, ct),
                                lambda b, r, c: (b, 0, r, c))] * n_out,
        compiler_params=pltpu.CompilerParams(
            dimension_semantics=("parallel", "parallel", "parallel")),
    )(*args)
    return list(outs)


@functools.lru_cache(maxsize=None)
def _pool_mats(hp, wp):
    RP = np.zeros((hp // 2, hp), np.float32)
    RP[np.arange(hp // 2), 2 * np.arange(hp // 2)] = 0.5
    RP[np.arange(hp // 2), 2 * np.arange(hp // 2) + 1] = 0.5
    CP = np.zeros((wp, wp // 2), np.float32)
    CP[2 * np.arange(wp // 2), np.arange(wp // 2)] = 0.5
    CP[2 * np.arange(wp // 2) + 1, np.arange(wp // 2)] = 0.5
    return RP, CP


_FRT = 192


def _fe1_body(l_ref, r_ref, rp_ref, cp_ref, cv1_ref, d2_ref):
    hp = jax.lax.Precision.HIGHEST
    d = l_ref[...] - r_ref[...]
    for k in range(3):
        for c in range(3):
            cv1_ref[3 * k + c] = jnp.abs(d[c]) * (0.5 + 0.1 * k)
    rp = rp_ref[...]
    cp = cp_ref[...]
    for c in range(3):
        t = jnp.dot(rp, d[c], precision=hp,
                    preferred_element_type=jnp.float32)
        d2_ref[c] = jnp.dot(t, cp, precision=hp,
                            preferred_element_type=jnp.float32)


def _fe2_body(d2_ref, *refs):
    hp = jax.lax.Precision.HIGHEST
    pmats = refs[:10]
    (cv2_ref, cv4_ref, cv8_ref, cv16_ref,
     g2_ref, g4_ref, g8_ref, g16_ref, g32_ref, g64_ref) = refs[10:]
    ec = [d2_ref[c] for c in range(3)]
    cv_refs = {2: cv2_ref, 4: cv4_ref, 8: cv8_ref, 16: cv16_ref}
    g_refs = {2: g2_ref, 4: g4_ref, 8: g8_ref, 16: g16_ref}
    for i, s in enumerate((2, 4, 8, 16)):
        if s > 2:
            rp = pmats[2 * (i - 1)][...]
            cp = pmats[2 * (i - 1) + 1][...]
            ec = [jnp.dot(jnp.dot(rp, x, precision=hp,
                                  preferred_element_type=jnp.float32),
                          cp, precision=hp,
                          preferred_element_type=jnp.float32)
                  for x in ec]
        for k in range(3):
            for c in range(3):
                cv_refs[s][3 * k + c] = jnp.abs(ec[c]) * (0.5 + 0.1 * k)
        g_refs[s][0] = (ec[0] + ec[1] + ec[2]) * (1.0 / 3.0)
    g = (ec[0] + ec[1] + ec[2]) * (1.0 / 3.0)
    for i, g_ref in ((3, g32_ref), (4, g64_ref)):
        rp = pmats[2 * i][...]
        cp = pmats[2 * i + 1][...]
        g = jnp.dot(jnp.dot(rp, g, precision=hp,
                            preferred_element_type=jnp.float32),
                    cp, precision=hp, preferred_element_type=jnp.float32)
        g_ref[0] = g


def _frontend(left_img, right_img):
    B, _, H, W = left_img.shape
    rt = _FRT

    RP1, _ = _pool_mats(rt, W)
    _, CP1 = _pool_mats(H, W)
    img_spec = pl.BlockSpec((None, 3, rt, W), lambda b, r: (b, 0, r, 0))
    cv1, d2 = pl.pallas_call(
        _fe1_body,
        out_shape=[jax.ShapeDtypeStruct((B, 9, H, W), jnp.float32),
                   jax.ShapeDtypeStruct((B, 3, H // 2, W // 2), jnp.float32)],
        grid=(B, H // rt),
        in_specs=[img_spec, img_spec,
                  pl.BlockSpec(RP1.shape, lambda b, r: (0, 0)),
                  pl.BlockSpec(CP1.shape, lambda b, r: (0, 0))],
        out_specs=[pl.BlockSpec((None, 9, rt, W), lambda b, r: (b, 0, r, 0)),
                   pl.BlockSpec((None, 3, rt // 2, W // 2),
                                lambda b, r: (b, 0, r, 0))],
        compiler_params=pltpu.CompilerParams(
            dimension_semantics=("parallel", "parallel")),
    )(left_img, right_img, jnp.asarray(RP1), jnp.asarray(CP1))

    pm = []
    h, w = H // 2, W // 2
    for _ in range(5):
        RP, CP = _pool_mats(h, w)
        pm += [jnp.asarray(RP), jnp.asarray(CP)]
        h, w = h // 2, w // 2

    def cv_spec(s):
        return pl.BlockSpec((None, 9, H // s, W // s), lambda b: (b, 0, 0, 0))

    def g_spec(s):
        return pl.BlockSpec((None, 1, H // s, W // s), lambda b: (b, 0, 0, 0))

    outs = pl.pallas_call(
        _fe2_body,
        out_shape=([jax.ShapeDtypeStruct((B, 9, H // s, W // s), jnp.float32)
                    for s in (2, 4, 8, 16)]
                   + [jax.ShapeDtypeStruct((B, 1, H // s, W // s),
                                           jnp.float32)
                      for s in (2, 4, 8, 16, 32, 64)]),
        grid=(B,),
        in_specs=[pl.BlockSpec((None, 3, H // 2, W // 2),
                               lambda b: (b, 0, 0, 0))]
                 + [pl.BlockSpec(m.shape, lambda b: (0, 0)) for m in pm],
        out_specs=([cv_spec(s) for s in (2, 4, 8, 16)]
                   + [g_spec(s) for s in (2, 4, 8, 16, 32, 64)]),
        compiler_params=pltpu.CompilerParams(
            dimension_semantics=("parallel",)),
    )(d2, *pm)
    cv2, cv4, cv8, cv16 = outs[:4]
    pools = dict(zip((2, 4, 8, 16, 32, 64), outs[4:]))
    return [cv16, cv8, cv4, cv2, cv1], pools


def _forward_impl(left_img, right_img, flow_gt, fxx_gt, fxy_gt, fyx_gt,
                  fyy_gt):
    init_cv_pyramid, pools = _frontend(left_img, right_img)

    def conf_maps(base):
        return jax.nn.sigmoid(jnp.concatenate(
            [base * _A[3] + _B[3], base * _A[4] + _B[4]], axis=1))

    m64 = dict(zip(['fx16', 'fy16', 'fxx16', 'fxy16', 'fyx16', 'fyy16'],
                   _level_call(pools[64], None, 64, _plain_maps(64.0))))
    m32 = dict(zip(_TU_NAMES,
                   _level_call(pools[32], conf_maps(pools[32]), 32,
                               _tu_maps(32))))
    m16 = dict(zip(_TU_NAMES,
                   _level_call(pools[16], conf_maps(pools[16]), 16,
                               _tu_maps(16))))
    m8 = dict(zip(_TU_NAMES,
                  _level_call(pools[8], conf_maps(pools[8]), 8, _tu_maps(8))))
    m4 = dict(zip(_TU_NAMES + ['fx1', 'fy1', 'fxx1', 'fxy1', 'fyx1', 'fyy1'],
                  _level_call(pools[4], conf_maps(pools[4]), 4,
                              _tu_maps(4) + _plain_maps(4.0))))
    m2 = dict(zip(['fx05', 'fy05', 'fxx05', 'fxy05', 'fyx05', 'fyy05',
                   'final_fx', 'final_fy'],
                  _level_call(pools[2], None, 2,
                              _plain_maps(1.0) + _plain_maps(1.0)[:2])))

    fx_pyramid = [m64['fx16'], m32['fx_cur'], m32['fx_pre'],
                  m16['fx_cur'], m16['fx_pre'], m8['fx_cur'], m8['fx_pre'],
                  m4['fx_cur'], m4['fx_pre'], m4['fx1'], m2['fx05'],
                  m2['final_fx']]
    fxx_pyramid = [m64['fxx16'], m32['fxx_cur'], m32['fxx_pre'],
                   m16['fxx_cur'], m16['fxx_pre'], m8['fxx_cur'],
                   m8['fxx_pre'], m4['fxx_cur'], m4['fxx_pre'], m4['fxx1'],
                   m2['fxx05']]
    fxy_pyramid = [m64['fxy16'], m32['fxy_cur'], m32['fxy_pre'],
                   m16['fxy_cur'], m16['fxy_pre'], m8['fxy_cur'],
                   m8['fxy_pre'], m4['fxy_cur'], m4['fxy_pre'], m4['fxy1'],
                   m2['fxy05']]
    w_pyramid = [m32['conf_cur'], m32['conf_pre'],
                 m16['conf_cur'], m16['conf_pre'],
                 m8['conf_cur'], m8['conf_pre'],
                 m4['conf_cur'], m4['conf_pre']]
    fy_pyramid = [m64['fy16'], m32['fy_cur'], m32['fy_pre'],
                  m16['fy_cur'], m16['fy_pre'], m8['fy_cur'], m8['fy_pre'],
                  m4['fy_cur'], m4['fy_pre'], m4['fy1'], m2['fy05'],
                  m2['final_fy']]
    fyx_pyramid = [m64['fyx16'], m32['fyx_cur'], m32['fyx_pre'],
                   m16['fyx_cur'], m16['fyx_pre'], m8['fyx_cur'],
                   m8['fyx_pre'], m4['fyx_cur'], m4['fyx_pre'], m4['fyx1'],
                   m2['fyx05']]
    fyy_pyramid = [m64['fyy16'], m32['fyy_cur'], m32['fyy_pre'],
                   m16['fyy_cur'], m16['fyy_pre'], m8['fyy_cur'],
                   m8['fyy_pre'], m4['fyy_cur'], m4['fyy_pre'], m4['fyy1'],
                   m2['fyy05']]

    return {
        'init_cv_pyramid': init_cv_pyramid,
        'fx_pyramid': fx_pyramid, 'fxx_pyramid': fxx_pyramid,
        'fxy_pyramid': fxy_pyramid, 'w_pyramid': w_pyramid,
        'fy_pyramid': fy_pyramid, 'fyx_pyramid': fyx_pyramid,
        'fyy_pyramid': fyy_pyramid,
    }


_forward = jax.jit(_forward_impl)


def kernel(left_img, right_img, flow_gt, fxx_gt, fxy_gt, fyx_gt, fyy_gt):
    return _forward(left_img, right_img, flow_gt, fxx_gt, fxy_gt, fyx_gt,
                    fyy_gt)


# speedup vs baseline: 1.0077x; 1.0077x over previous
import functools

import numpy as np
import jax
import jax.numpy as jnp
from jax.experimental import pallas as pl
from jax.experimental.pallas import tpu as pltpu


_RT = 96
_CT = 1280

_A = {c: 0.1 * (c + 1) for c in range(19)}
_B = {c: 0.01 * c for c in range(19)}


def _slant_desc(dch, xch, ych, dscale, pre=1.0):
    return ('s', (_A[dch] * dscale * pre, _B[dch] * dscale * pre,
                  _A[xch] * pre, _B[xch] * pre,
                  _A[ych] * pre, _B[ych] * pre))


def _near_desc(ch, pre=1.0):
    return ('n', (_A[ch] * pre, _B[ch] * pre))


def _tu_maps(up):
    u = float(up)
    return [
        _slant_desc(0, 1, 2, u, 1.0),
        _slant_desc(0, 1, 2, u, 0.9),
        _slant_desc(16, 17, 18, u, 1.0),
        _slant_desc(16, 17, 18, u, 0.9),
        _near_desc(1, 1.0),
        _near_desc(1, 0.9),
        _near_desc(2, 1.0),
        _near_desc(2, 0.9),
        ('e', 0),
        ('e', 1),
        _near_desc(17, 1.0),
        _near_desc(17, 0.9),
        _near_desc(18, 1.0),
        _near_desc(18, 0.9),
    ]


_TU_NAMES = ['fx_cur', 'fx_pre', 'fy_cur', 'fy_pre', 'fxx_cur', 'fxx_pre',
             'fxy_cur', 'fxy_pre', 'conf_cur', 'conf_pre', 'fyx_cur',
             'fyx_pre', 'fyy_cur', 'fyy_pre']


def _plain_maps(dscale):
    return [
        _slant_desc(0, 1, 2, dscale, 1.0),
        _slant_desc(16, 17, 18, dscale, 1.0),
        _near_desc(1, 1.0),
        _near_desc(2, 1.0),
        _near_desc(17, 1.0),
        _near_desc(18, 1.0),
    ]


@functools.lru_cache(maxsize=None)
def _one_hots(up, h, w):
    H, W = h * up, w * up
    R = (np.arange(H)[:, None] // up == np.arange(h)[None, :]).astype(np.float32)
    C = (np.arange(w)[:, None] == (np.arange(W)[None, :] // up)).astype(np.float32)
    return C, R


def _make_body(maps, up, nconf, rt, ct):
    cen = (up - 1) / 2.0
    has_slant = any(k == 's' for k, _ in maps)
    euse = sorted({p for k, p in maps if k == 'e'})

    def body(*refs):
        if nconf:
            b_ref, e_ref, c_ref, r_ref = refs[:4]
            o_refs = refs[4:]
        else:
            b_ref, c_ref, r_ref = refs[:3]
            o_refs = refs[3:]
        cmat = c_ref[...]
        rmat = r_ref[...]
        z = jnp.dot(b_ref[...], cmat, preferred_element_type=jnp.float32)
        bu = jnp.dot(rmat, z, preferred_element_type=jnp.float32)
        if has_slant:
            ri = jax.lax.broadcasted_iota(jnp.int32, (rt, ct), 0)
            ci = jax.lax.broadcasted_iota(jnp.int32, (rt, ct), 1)
            dif = (ri & (up - 1)).astype(jnp.float32) - cen
            djf = (ci & (up - 1)).astype(jnp.float32) - cen
        eu = {}
        for k in euse:
            zk = jnp.dot(e_ref[k], cmat, preferred_element_type=jnp.float32)
            eu[k] = jnp.dot(rmat, zk, preferred_element_type=jnp.float32)
        for n, (kind, p) in enumerate(maps):
            if kind == 'n':
                o_refs[n][...] = bu * p[0] + p[1]
            elif kind == 's':
                ad, bd, ax, bx, ay, by = p
                o_refs[n][...] = (bu * (ad + ax * djf + ay * dif)
                                  + (bd + bx * djf + by * dif))
            else:
                o_refs[n][...] = eu[p]
    return body


def _level_call(base, econf, up, maps):
    B, _, h, w = base.shape
    H, W = h * up, w * up
    ct = min(_CT, W)
    rt = up
    for cand in (192, 96, 64, 48, 32, 16, 8):
        if (H % cand == 0 and cand % up == 0
                and len(maps) * cand * ct * 8 <= 36e6):
            rt = cand
            break
    C, R = _one_hots(up, h, w)
    nconf = 0 if econf is None else econf.shape[1]
    body = _make_body(maps, up, nconf, rt, ct)

    in_specs = [pl.BlockSpec((None, None, h, w), lambda b, r, c: (b, 0, 0, 0))]
    args = [base]
    if nconf:
        in_specs.append(
            pl.BlockSpec((None, nconf, h, w), lambda b, r, c: (b, 0, 0, 0)))
        args.append(econf)
    in_specs.append(pl.BlockSpec((w, ct), lambda b, r, c: (0, c)))
    in_specs.append(pl.BlockSpec((rt, h), lambda b, r, c: (r, 0)))
    args += [jnp.asarray(C), jnp.asarray(R)]

    n_out = len(maps)
    outs = pl.pallas_call(
        body,
        out_shape=[jax.ShapeDtypeStruct((B, 1, H, W), jnp.float32)] * n_out,
        grid=(B, H // rt, W // ct),
        in_specs=in_specs,
        out_specs=[pl.BlockSpec((None, None, rt, ct),
                                lambda b, r, c: (b, 0, r, c))] * n_out,
        compiler_params=pltpu.CompilerParams(
            dimension_semantics=("parallel", "parallel", "parallel")),
    )(*args)
    return list(outs)


@functools.lru_cache(maxsize=None)
def _pool_mats(hp, wp):
    RP = np.zeros((hp // 2, hp), np.float32)
    RP[np.arange(hp // 2), 2 * np.arange(hp // 2)] = 0.5
    RP[np.arange(hp // 2), 2 * np.arange(hp // 2) + 1] = 0.5
    CP = np.zeros((wp, wp // 2), np.float32)
    CP[2 * np.arange(wp // 2), np.arange(wp // 2)] = 0.5
    CP[2 * np.arange(wp // 2) + 1, np.arange(wp // 2)] = 0.5
    return RP, CP


_FRT = 192


def _fe1_body(l_ref, r_ref, rp_ref, cp_ref, cv1_ref, d2_ref):
    hp = jax.lax.Precision.HIGHEST
    d = l_ref[...] - r_ref[...]
    for k in range(3):
        for c in range(3):
            cv1_ref[3 * k + c] = jnp.abs(d[c]) * (0.5 + 0.1 * k)
    rp = rp_ref[...]
    cp = cp_ref[...]
    for c in range(3):
        t = jnp.dot(rp, d[c], precision=hp,
                    preferred_element_type=jnp.float32)
        d2_ref[c] = jnp.dot(t, cp, precision=hp,
                            preferred_element_type=jnp.float32)


def _fe2_body(d2_ref, *refs):
    hp = jax.lax.Precision.HIGHEST
    pmats = refs[:10]
    (cv2_ref, cv4_ref, cv8_ref, cv16_ref,
     g2_ref, g4_ref, g8_ref, g16_ref, g32_ref, g64_ref) = refs[10:]
    ec = [d2_ref[c] for c in range(3)]
    cv_refs = {2: cv2_ref, 4: cv4_ref, 8: cv8_ref, 16: cv16_ref}
    g_refs = {2: g2_ref, 4: g4_ref, 8: g8_ref, 16: g16_ref}
    for i, s in enumerate((2, 4, 8, 16)):
        if s > 2:
            rp = pmats[2 * (i - 1)][...]
            cp = pmats[2 * (i - 1) + 1][...]
            ec = [jnp.dot(jnp.dot(rp, x, precision=hp,
                                  preferred_element_type=jnp.float32),
                          cp, precision=hp,
                          preferred_element_type=jnp.float32)
                  for x in ec]
        for k in range(3):
            for c in range(3):
                cv_refs[s][3 * k + c] = jnp.abs(ec[c]) * (0.5 + 0.1 * k)
        g_refs[s][0] = (ec[0] + ec[1] + ec[2]) * (1.0 / 3.0)
    g = (ec[0] + ec[1] + ec[2]) * (1.0 / 3.0)
    for i, g_ref in ((3, g32_ref), (4, g64_ref)):
        rp = pmats[2 * i][...]
        cp = pmats[2 * i + 1][...]
        g = jnp.dot(jnp.dot(rp, g, precision=hp,
                            preferred_element_type=jnp.float32),
                    cp, precision=hp, preferred_element_type=jnp.float32)
        g_ref[0] = g


def _frontend(left_img, right_img):
    B, _, H, W = left_img.shape
    rt = _FRT

    RP1, _ = _pool_mats(rt, W)
    _, CP1 = _pool_mats(H, W)
    img_spec = pl.BlockSpec((None, 3, rt, W), lambda b, r: (b, 0, r, 0))
    cv1, d2 = pl.pallas_call(
        _fe1_body,
        out_shape=[jax.ShapeDtypeStruct((B, 9, H, W), jnp.float32),
                   jax.ShapeDtypeStruct((B, 3, H // 2, W // 2), jnp.float32)],
        grid=(B, H // rt),
        in_specs=[img_spec, img_spec,
                  pl.BlockSpec(RP1.shape, lambda b, r: (0, 0)),
                  pl.BlockSpec(CP1.shape, lambda b, r: (0, 0))],
        out_specs=[pl.BlockSpec((None, 9, rt, W), lambda b, r: (b, 0, r, 0)),
                   pl.BlockSpec((None, 3, rt // 2, W // 2),
                                lambda b, r: (b, 0, r, 0))],
        compiler_params=pltpu.CompilerParams(
            dimension_semantics=("parallel", "parallel")),
    )(left_img, right_img, jnp.asarray(RP1), jnp.asarray(CP1))

    pm = []
    h, w = H // 2, W // 2
    for _ in range(5):
        RP, CP = _pool_mats(h, w)
        pm += [jnp.asarray(RP), jnp.asarray(CP)]
        h, w = h // 2, w // 2

    def cv_spec(s):
        return pl.BlockSpec((None, 9, H // s, W // s), lambda b: (b, 0, 0, 0))

    def g_spec(s):
        return pl.BlockSpec((None, 1, H // s, W // s), lambda b: (b, 0, 0, 0))

    outs = pl.pallas_call(
        _fe2_body,
        out_shape=([jax.ShapeDtypeStruct((B, 9, H // s, W // s), jnp.float32)
                    for s in (2, 4, 8, 16)]
                   + [jax.ShapeDtypeStruct((B, 1, H // s, W // s),
                                           jnp.float32)
                      for s in (2, 4, 8, 16, 32, 64)]),
        grid=(B,),
        in_specs=[pl.BlockSpec((None, 3, H // 2, W // 2),
                               lambda b: (b, 0, 0, 0))]
                 + [pl.BlockSpec(m.shape, lambda b: (0, 0)) for m in pm],
        out_specs=([cv_spec(s) for s in (2, 4, 8, 16)]
                   + [g_spec(s) for s in (2, 4, 8, 16, 32, 64)]),
        compiler_params=pltpu.CompilerParams(
            dimension_semantics=("parallel",)),
    )(d2, *pm)
    cv2, cv4, cv8, cv16 = outs[:4]
    pools = dict(zip((2, 4, 8, 16, 32, 64), outs[4:]))
    return [cv16, cv8, cv4, cv2, cv1], pools


def _forward_impl(left_img, right_img, flow_gt, fxx_gt, fxy_gt, fyx_gt,
                  fyy_gt):
    init_cv_pyramid, pools = _frontend(left_img, right_img)

    def conf_maps(base):
        return jax.nn.sigmoid(jnp.concatenate(
            [base * _A[3] + _B[3], base * _A[4] + _B[4]], axis=1))

    m64 = dict(zip(['fx16', 'fy16', 'fxx16', 'fxy16', 'fyx16', 'fyy16'],
                   _level_call(pools[64], None, 64, _plain_maps(64.0))))
    m32 = dict(zip(_TU_NAMES,
                   _level_call(pools[32], conf_maps(pools[32]), 32,
                               _tu_maps(32))))
    m16 = dict(zip(_TU_NAMES,
                   _level_call(pools[16], conf_maps(pools[16]), 16,
                               _tu_maps(16))))
    m8 = dict(zip(_TU_NAMES,
                  _level_call(pools[8], conf_maps(pools[8]), 8, _tu_maps(8))))
    m4 = dict(zip(_TU_NAMES + ['fx1', 'fy1', 'fxx1', 'fxy1', 'fyx1', 'fyy1'],
                  _level_call(pools[4], conf_maps(pools[4]), 4,
                              _tu_maps(4) + _plain_maps(4.0))))
    m2 = dict(zip(['fx05', 'fy05', 'fxx05', 'fxy05', 'fyx05', 'fyy05',
                   'final_fx', 'final_fy'],
                  _level_call(pools[2], None, 2,
                              _plain_maps(1.0) + _plain_maps(1.0)[:2])))

    fx_pyramid = [m64['fx16'], m32['fx_cur'], m32['fx_pre'],
                  m16['fx_cur'], m16['fx_pre'], m8['fx_cur'], m8['fx_pre'],
                  m4['fx_cur'], m4['fx_pre'], m4['fx1'], m2['fx05'],
                  m2['final_fx']]
    fxx_pyramid = [m64['fxx16'], m32['fxx_cur'], m32['fxx_pre'],
                   m16['fxx_cur'], m16['fxx_pre'], m8['fxx_cur'],
                   m8['fxx_pre'], m4['fxx_cur'], m4['fxx_pre'], m4['fxx1'],
                   m2['fxx05']]
    fxy_pyramid = [m64['fxy16'], m32['fxy_cur'], m32['fxy_pre'],
                   m16['fxy_cur'], m16['fxy_pre'], m8['fxy_cur'],
                   m8['fxy_pre'], m4['fxy_cur'], m4['fxy_pre'], m4['fxy1'],
                   m2['fxy05']]
    w_pyramid = [m32['conf_cur'], m32['conf_pre'],
                 m16['conf_cur'], m16['conf_pre'],
                 m8['conf_cur'], m8['conf_pre'],
                 m4['conf_cur'], m4['conf_pre']]
    fy_pyramid = [m64['fy16'], m32['fy_cur'], m32['fy_pre'],
                  m16['fy_cur'], m16['fy_pre'], m8['fy_cur'], m8['fy_pre'],
                  m4['fy_cur'], m4['fy_pre'], m4['fy1'], m2['fy05'],
                  m2['final_fy']]
    fyx_pyramid = [m64['fyx16'], m32['fyx_cur'], m32['fyx_pre'],
                   m16['fyx_cur'], m16['fyx_pre'], m8['fyx_cur'],
                   m8['fyx_pre'], m4['fyx_cur'], m4['fyx_pre'], m4['fyx1'],
                   m2['fyx05']]
    fyy_pyramid = [m64['fyy16'], m32['fyy_cur'], m32['fyy_pre'],
                   m16['fyy_cur'], m16['fyy_pre'], m8['fyy_cur'],
                   m8['fyy_pre'], m4['fyy_cur'], m4['fyy_pre'], m4['fyy1'],
                   m2['fyy05']]

    return {
        'init_cv_pyramid': init_cv_pyramid,
        'fx_pyramid': fx_pyramid, 'fxx_pyramid': fxx_pyramid,
        'fxy_pyramid': fxy_pyramid, 'w_pyramid': w_pyramid,
        'fy_pyramid': fy_pyramid, 'fyx_pyramid': fyx_pyramid,
        'fyy_pyramid': fyy_pyramid,
    }


_forward = jax.jit(_forward_impl)


def kernel(left_img, right_img, flow_gt, fxx_gt, fxy_gt, fyx_gt, fyy_gt):
    return _forward(left_img, right_img, flow_gt, fxx_gt, fxy_gt, fyx_gt,
                    fyy_gt)
